# baseline (speedup 1.0000x reference)
import functools

import jax
import jax.numpy as jnp
from jax import lax
from jax.experimental import pallas as pl
from jax.experimental.pallas import tpu as pltpu
from jax.experimental.pallas import tpu_sc as plsc

_TOPK = 8
_N_EXPERTS = 64
_ROUTED_SCALE = 2.5
_NC = 2
_NS = 16
_NW = _NC * _NS


def _matmul_body(x_ref, wt_ref, s_ref):
    s_ref[...] = jnp.dot(
        x_ref[...], wt_ref[...], preferred_element_type=jnp.float32
    )


def _scores_tc(x, wt, blk):
    tokens, dim = x.shape
    return pl.pallas_call(
        _matmul_body,
        grid=(tokens // blk,),
        in_specs=[
            pl.BlockSpec((blk, dim), lambda i: (i, 0)),
            pl.BlockSpec((dim, _N_EXPERTS), lambda i: (0, 0)),
        ],
        out_specs=pl.BlockSpec((blk, _N_EXPERTS), lambda i: (i, 0)),
        out_shape=jax.ShapeDtypeStruct((tokens, _N_EXPERTS), jnp.float32),
    )(x, wt)


def _make_sc_topk(tokens):
    tpw = tokens // _NW
    mesh = plsc.VectorSubcoreMesh(
        core_axis_name="c", subcore_axis_name="s",
        num_cores=_NC, num_subcores=_NS,
    )

    @functools.partial(
        pl.kernel,
        out_type=[
            jax.ShapeDtypeStruct((tokens * _TOPK,), jnp.int32),
            jax.ShapeDtypeStruct((tokens * _TOPK,), jnp.float32),
        ],
        mesh=mesh,
        compiler_params=pltpu.CompilerParams(needs_layout_passes=False),
        scratch_types=[
            pltpu.VMEM((tpw * _N_EXPERTS,), jnp.float32),
            pltpu.VMEM((tpw * _TOPK + 8,), jnp.int32),
            pltpu.VMEM((tpw * _TOPK + 8,), jnp.float32),
        ],
    )
    def sc_topk(scores_hbm, idx_hbm, w_hbm, sv, iv, wv):
        wid = lax.axis_index("s") * _NC + lax.axis_index("c")
        pltpu.sync_copy(scores_hbm.at[pl.ds(wid * tpw * _N_EXPERTS, tpw * _N_EXPERTS)], sv)
        lane = lax.iota(jnp.int32, 16)
        lo8 = lane < 8

        def merge(ak, ai, bk, bi):
            ck = jnp.where(lo8, ak, jnp.flip(bk))
            ci = jnp.where(lo8, ai, jnp.flip(bi))
            return plsc.sort_key_val(ck, ci, descending=True)

        def body(t, carry):
            off = t * _N_EXPERTS
            ks = []
            js = []
            for c in range(4):
                k, i = plsc.sort_key_val(
                    sv[pl.ds(off + 16 * c, 16)], lane + 16 * c, descending=True
                )
                ks.append(k)
                js.append(i)
            m0k, m0i = merge(ks[0], js[0], ks[1], js[1])
            m1k, m1i = merge(ks[2], js[2], ks[3], js[3])
            fk, fi = merge(m0k, m0i, m1k, m1i)
            w = 1.0 / (1.0 + jnp.exp(-fk))
            total = jnp.sum(jnp.where(lo8, w, 0.0), axis=0)
            w = (w * _ROUTED_SCALE) / jnp.broadcast_to(total, (16,))
            plsc.store_compressed(iv.at[pl.ds(t * _TOPK, 16)], fi, mask=lo8)
            plsc.store_compressed(wv.at[pl.ds(t * _TOPK, 16)], w, mask=lo8)
            return carry

        lax.fori_loop(0, tpw, body, 0)
        pltpu.sync_copy(iv.at[pl.ds(0, tpw * _TOPK)], idx_hbm.at[pl.ds(wid * tpw * _TOPK, tpw * _TOPK)])
        pltpu.sync_copy(wv.at[pl.ds(0, tpw * _TOPK)], w_hbm.at[pl.ds(wid * tpw * _TOPK, tpw * _TOPK)])

    return sc_topk


@functools.partial(jax.jit, static_argnames=("blk",))
def _gate(x, weight, blk=1024):
    tokens = x.shape[0]
    scores = _scores_tc(x, weight.T, blk)
    idx_flat, w_flat = _make_sc_topk(tokens)(scores.reshape(-1))
    return idx_flat.reshape(tokens, _TOPK), w_flat.reshape(tokens, _TOPK)


def kernel(x, weight):
    return _gate(x, weight)


# speedup vs baseline: 1.2185x; 1.2185x over previous
import functools

import jax
import jax.numpy as jnp
from jax import lax
from jax.experimental import pallas as pl
from jax.experimental.pallas import tpu as pltpu
from jax.experimental.pallas import tpu_sc as plsc

_TOPK = 8
_N_EXPERTS = 64
_ROUTED_SCALE = 2.5
_NC = 2
_NS = 16
_NW = _NC * _NS


def _matmul_body(x_ref, wt_ref, s_ref):
    s_ref[...] = jnp.dot(
        x_ref[...], wt_ref[...], preferred_element_type=jnp.float32
    )


def _scores_tc(x, wt, blk):
    tokens, dim = x.shape
    return pl.pallas_call(
        _matmul_body,
        grid=(tokens // blk,),
        in_specs=[
            pl.BlockSpec((blk, dim), lambda i: (i, 0)),
            pl.BlockSpec((dim, _N_EXPERTS), lambda i: (0, 0)),
        ],
        out_specs=pl.BlockSpec((blk, _N_EXPERTS), lambda i: (i, 0)),
        out_shape=jax.ShapeDtypeStruct((tokens, _N_EXPERTS), jnp.float32),
    )(x, wt)


def _make_sc_topk(tokens):
    tpw = tokens // _NW
    mesh = plsc.VectorSubcoreMesh(
        core_axis_name="c", subcore_axis_name="s",
        num_cores=_NC, num_subcores=_NS,
    )

    @functools.partial(
        pl.kernel,
        out_type=[
            jax.ShapeDtypeStruct((tokens * _TOPK,), jnp.int32),
            jax.ShapeDtypeStruct((tokens * _TOPK,), jnp.float32),
        ],
        mesh=mesh,
        compiler_params=pltpu.CompilerParams(needs_layout_passes=False),
        scratch_types=[
            pltpu.VMEM((tpw * _N_EXPERTS,), jnp.float32),
            pltpu.VMEM((tpw * _TOPK + 8,), jnp.int32),
            pltpu.VMEM((tpw * _TOPK + 8,), jnp.float32),
        ],
    )
    def sc_topk(scores_hbm, idx_hbm, w_hbm, sv, iv, wv):
        wid = lax.axis_index("s") * _NC + lax.axis_index("c")
        pltpu.sync_copy(scores_hbm.at[pl.ds(wid * tpw * _N_EXPERTS, tpw * _N_EXPERTS)], sv)
        lane = lax.iota(jnp.int32, 16)
        lo8 = lane < 8

        def merge(ak, ai, bk, bi):
            ck = jnp.where(lo8, ak, jnp.flip(bk))
            ci = jnp.where(lo8, ai, jnp.flip(bi))
            return plsc.sort_key_val(ck, ci, descending=True)

        @plsc.parallel_loop(0, tpw, unroll=4)
        def body(t):
            off = t * _N_EXPERTS
            ks = []
            js = []
            for c in range(4):
                k, i = plsc.sort_key_val(
                    sv[pl.ds(off + 16 * c, 16)], lane + 16 * c, descending=True
                )
                ks.append(k)
                js.append(i)
            m0k, m0i = merge(ks[0], js[0], ks[1], js[1])
            m1k, m1i = merge(ks[2], js[2], ks[3], js[3])
            fk, fi = merge(m0k, m0i, m1k, m1i)
            w = 1.0 / (1.0 + jnp.exp(-fk))
            total = jnp.sum(jnp.where(lo8, w, 0.0), axis=0)
            w = (w * _ROUTED_SCALE) / jnp.broadcast_to(total, (16,))
            plsc.store_compressed(iv.at[pl.ds(t * _TOPK, 16)], fi, mask=lo8)
            plsc.store_compressed(wv.at[pl.ds(t * _TOPK, 16)], w, mask=lo8)

        pltpu.sync_copy(iv.at[pl.ds(0, tpw * _TOPK)], idx_hbm.at[pl.ds(wid * tpw * _TOPK, tpw * _TOPK)])
        pltpu.sync_copy(wv.at[pl.ds(0, tpw * _TOPK)], w_hbm.at[pl.ds(wid * tpw * _TOPK, tpw * _TOPK)])

    return sc_topk


@functools.partial(jax.jit, static_argnames=("blk",))
def _gate(x, weight, blk=1024):
    tokens = x.shape[0]
    scores = _scores_tc(x, weight.T, blk)
    idx_flat, w_flat = _make_sc_topk(tokens)(scores.reshape(-1))
    return idx_flat.reshape(tokens, _TOPK), w_flat.reshape(tokens, _TOPK)


def kernel(x, weight):
    return _gate(x, weight)


# speedup vs baseline: 1.2601x; 1.0341x over previous
import functools

import jax
import jax.numpy as jnp
from jax import lax
from jax.experimental import pallas as pl
from jax.experimental.pallas import tpu as pltpu
from jax.experimental.pallas import tpu_sc as plsc

_TOPK = 8
_N_EXPERTS = 64
_ROUTED_SCALE = 2.5
_NC = 2
_NS = 16
_NW = _NC * _NS
_CHUNKS = 4


def _matmul_body(x_ref, wt_ref, s_ref):
    s_ref[...] = jnp.dot(
        x_ref[...], wt_ref[...], preferred_element_type=jnp.float32
    )


def _scores_tc(x, wt, chunk, blk, row0):
    dim = x.shape[1]
    return pl.pallas_call(
        _matmul_body,
        grid=(chunk // blk,),
        in_specs=[
            pl.BlockSpec((blk, dim), lambda i: (row0 // blk + i, 0)),
            pl.BlockSpec((dim, _N_EXPERTS), lambda i: (0, 0)),
        ],
        out_specs=pl.BlockSpec((blk, _N_EXPERTS), lambda i: (i, 0)),
        out_shape=jax.ShapeDtypeStruct((chunk, _N_EXPERTS), jnp.float32),
    )(x, wt)


def _make_sc_topk(tokens):
    tpw = tokens // _NW
    mesh = plsc.VectorSubcoreMesh(
        core_axis_name="c", subcore_axis_name="s",
        num_cores=_NC, num_subcores=_NS,
    )

    @functools.partial(
        pl.kernel,
        out_type=[
            jax.ShapeDtypeStruct((tokens * _TOPK,), jnp.int32),
            jax.ShapeDtypeStruct((tokens * _TOPK,), jnp.float32),
        ],
        mesh=mesh,
        compiler_params=pltpu.CompilerParams(needs_layout_passes=False),
        scratch_types=[
            pltpu.VMEM((tpw, _N_EXPERTS), jnp.float32),
            pltpu.VMEM((tpw * _TOPK + 8,), jnp.int32),
            pltpu.VMEM((tpw * _TOPK + 8,), jnp.float32),
        ],
    )
    def sc_topk(scores_hbm, idx_hbm, w_hbm, sv, iv, wv):
        wid = lax.axis_index("s") * _NC + lax.axis_index("c")
        pltpu.sync_copy(scores_hbm.at[pl.ds(wid * tpw, tpw)], sv)
        lane = lax.iota(jnp.int32, 16)
        lo8 = lane < 8

        def merge(ak, ai, bk, bi):
            ck = jnp.where(lo8, ak, jnp.flip(bk))
            ci = jnp.where(lo8, ai, jnp.flip(bi))
            return plsc.sort_key_val(ck, ci, descending=True)

        @plsc.parallel_loop(0, tpw, unroll=4)
        def body(t):
            ks = []
            js = []
            for c in range(4):
                k, i = plsc.sort_key_val(
                    sv[t, pl.ds(16 * c, 16)], lane + 16 * c, descending=True
                )
                ks.append(k)
                js.append(i)
            m0k, m0i = merge(ks[0], js[0], ks[1], js[1])
            m1k, m1i = merge(ks[2], js[2], ks[3], js[3])
            fk, fi = merge(m0k, m0i, m1k, m1i)
            w = 1.0 / (1.0 + jnp.exp(-fk))
            total = jnp.sum(jnp.where(lo8, w, 0.0), axis=0)
            w = (w * _ROUTED_SCALE) / jnp.broadcast_to(total, (16,))
            plsc.store_compressed(iv.at[pl.ds(t * _TOPK, 16)], fi, mask=lo8)
            plsc.store_compressed(wv.at[pl.ds(t * _TOPK, 16)], w, mask=lo8)

        pltpu.sync_copy(iv.at[pl.ds(0, tpw * _TOPK)], idx_hbm.at[pl.ds(wid * tpw * _TOPK, tpw * _TOPK)])
        pltpu.sync_copy(wv.at[pl.ds(0, tpw * _TOPK)], w_hbm.at[pl.ds(wid * tpw * _TOPK, tpw * _TOPK)])

    return sc_topk


@functools.partial(jax.jit, static_argnames=("blk",))
def _gate(x, weight, blk=1024):
    tokens = x.shape[0]
    wt = weight.T
    chunk = tokens // _CHUNKS
    sc_topk = _make_sc_topk(chunk)
    idx_parts = []
    w_parts = []
    for c in range(_CHUNKS):
        scores_c = _scores_tc(x, wt, chunk, blk, c * chunk)
        idx_c, w_c = sc_topk(scores_c)
        idx_parts.append(idx_c.reshape(chunk, _TOPK))
        w_parts.append(w_c.reshape(chunk, _TOPK))
    return (
        jnp.concatenate(idx_parts, axis=0),
        jnp.concatenate(w_parts, axis=0),
    )


def kernel(x, weight):
    return _gate(x, weight)


# speedup vs baseline: 1.8610x; 1.4769x over previous
import functools

import jax
import jax.numpy as jnp
from jax.experimental import pallas as pl

_TOPK = 8
_N_EXPERTS = 64
_ROUTED_SCALE = 2.5


def _gate_body(x1_ref, x2_ref, wt1_ref, wt2_ref, idx_ref, w_ref):
    blk = x1_ref.shape[0]
    scores = jnp.dot(x1_ref[...], wt1_ref[...], preferred_element_type=jnp.float32)
    scores = scores + jnp.dot(x2_ref[...], wt2_ref[...], preferred_element_type=jnp.float32)
    st = scores.T
    iota = jax.lax.broadcasted_iota(jnp.int32, (_N_EXPERTS, blk), 0)
    vals = []
    idxs = []
    for _ in range(_TOPK):
        m = jnp.max(st, axis=0, keepdims=True)
        is_max = st == m
        sel = jnp.min(jnp.where(is_max, iota, _N_EXPERTS), axis=0, keepdims=True)
        vals.append(m)
        idxs.append(sel)
        st = jnp.where(iota == sel, -jnp.inf, st)
    v = jnp.concatenate(vals, axis=0)
    ix = jnp.concatenate(idxs, axis=0)
    v = 1.0 / (1.0 + jnp.exp(-v))
    v = v * (_ROUTED_SCALE / jnp.sum(v, axis=0, keepdims=True))
    idx_ref[...] = ix
    w_ref[...] = v


@functools.partial(jax.jit, static_argnames=("blk",))
def _gate(x, weight, blk=1024):
    tokens = x.shape[0]
    dim = x.shape[1]
    half = dim // 2
    wt = weight.T
    grid = (tokens // blk,)
    idx_t, w_t = pl.pallas_call(
        _gate_body,
        grid=grid,
        in_specs=[
            pl.BlockSpec((blk, half), lambda i: (i, 0)),
            pl.BlockSpec((blk, half), lambda i: (i, 1)),
            pl.BlockSpec((half, _N_EXPERTS), lambda i: (0, 0)),
            pl.BlockSpec((half, _N_EXPERTS), lambda i: (1, 0)),
        ],
        out_specs=[
            pl.BlockSpec((_TOPK, blk), lambda i: (0, i)),
            pl.BlockSpec((_TOPK, blk), lambda i: (0, i)),
        ],
        out_shape=[
            jax.ShapeDtypeStruct((_TOPK, tokens), jnp.int32),
            jax.ShapeDtypeStruct((_TOPK, tokens), jnp.float32),
        ],
    )(x, x, wt, wt)
    return idx_t.T, w_t.T


def kernel(x, weight):
    return _gate(x, weight)


# speedup vs baseline: 1.8667x; 1.0031x over previous
import functools

import jax
import jax.numpy as jnp
from jax.experimental import pallas as pl

_TOPK = 8
_N_EXPERTS = 64
_ROUTED_SCALE = 2.5


def _gate_body(x_ref, wt_ref, idx_ref, w_ref):
    blk = x_ref.shape[0]
    scores = jnp.dot(x_ref[...], wt_ref[...], preferred_element_type=jnp.float32)
    st = scores.T
    iota = jax.lax.broadcasted_iota(jnp.int32, (_N_EXPERTS, blk), 0)
    vals = []
    idxs = []
    for _ in range(_TOPK):
        m = jnp.max(st, axis=0, keepdims=True)
        is_max = st == m
        sel = jnp.min(jnp.where(is_max, iota, _N_EXPERTS), axis=0, keepdims=True)
        vals.append(m)
        idxs.append(sel)
        st = jnp.where(iota == sel, -jnp.inf, st)
    v = jnp.concatenate(vals, axis=0)
    ix = jnp.concatenate(idxs, axis=0)
    v = 1.0 / (1.0 + jnp.exp(-v))
    v = v * (_ROUTED_SCALE / jnp.sum(v, axis=0, keepdims=True))
    idx_ref[...] = ix
    w_ref[...] = v


@functools.partial(jax.jit, static_argnames=("blk",))
def _gate(x, weight, blk=1024):
    tokens = x.shape[0]
    dim = x.shape[1]
    wt = weight.T
    grid = (tokens // blk,)
    idx_t, w_t = pl.pallas_call(
        _gate_body,
        grid=grid,
        in_specs=[
            pl.BlockSpec((blk, dim), lambda i: (i, 0)),
            pl.BlockSpec((dim, _N_EXPERTS), lambda i: (0, 0)),
        ],
        out_specs=[
            pl.BlockSpec((_TOPK, blk), lambda i: (0, i)),
            pl.BlockSpec((_TOPK, blk), lambda i: (0, i)),
        ],
        out_shape=[
            jax.ShapeDtypeStruct((_TOPK, tokens), jnp.int32),
            jax.ShapeDtypeStruct((_TOPK, tokens), jnp.float32),
        ],
    )(x, wt)
    return idx_t.T, w_t.T


def kernel(x, weight):
    return _gate(x, weight)
